# baseline (speedup 1.0000x reference)
import functools

import jax
import jax.numpy as jnp
from jax import lax
from jax.experimental import pallas as pl
from jax.experimental.pallas import tpu as pltpu
from jax.experimental.pallas import tpu_sc as plsc

NS = 16
NC = 2
NQ = 2
K = 128
DQ = 128
RPT = 640
B_TC = 1024
NBUF = 2
NIDX = 2 * NBUF


@functools.lru_cache(maxsize=None)
def _sc_agg(n_rows, n_acc, n_chunks, do_deg):
    assert n_chunks % NIDX == 0
    n_g = n_chunks // NIDX
    half = n_chunks // 2
    out_ty = [jax.ShapeDtypeStruct((NQ, n_acc, DQ), jnp.float32)]
    if do_deg:
        out_ty.append(jax.ShapeDtypeStruct((NC, n_acc), jnp.float32))
    mesh = plsc.VectorSubcoreMesh(core_axis_name="c", subcore_axis_name="s")

    @functools.partial(
        pl.kernel,
        out_type=out_ty,
        mesh=mesh,
        scratch_types=[
            pltpu.VMEM((NIDX, K), jnp.int32),
            pltpu.VMEM((NIDX, K), jnp.int32),
            pltpu.VMEM((NBUF, K, DQ), jnp.float32),
            pltpu.VMEM((K,), jnp.float32),
            pltpu.VMEM_SHARED((n_acc, DQ), jnp.float32),
            pltpu.VMEM_SHARED((n_acc,), jnp.float32),
            pltpu.SemaphoreType.DMA((NIDX,)),
            pltpu.SemaphoreType.DMA((NBUF,)),
            pltpu.SemaphoreType.DMA((NBUF,)),
            pltpu.SemaphoreType.DMA((NBUF,)),
        ],
    )
    def kfn(tblq, srcp, dstp, zrows, zdeg, onesrow, *refs):
        if do_deg:
            (outagg, outdeg, src_v, dst_v, rows_v, ones_v, agg_sh,
             deg_sh, si, sg, ss, sd) = refs
        else:
            (outagg, src_v, dst_v, rows_v, ones_v, agg_sh,
             deg_sh, si, sg, ss, sd) = refs
        c = lax.axis_index("c")
        s = lax.axis_index("s")
        rbase = s * RPT
        ebase = s * (n_chunks * K)

        def fire_idx(j, slot):
            off = ebase + j * K
            pltpu.async_copy(srcp.at[pl.ds(off, K)], src_v.at[slot],
                             si.at[slot])
            pltpu.async_copy(dstp.at[pl.ds(off, K)], dst_v.at[slot],
                             si.at[slot])

        def wait_idx(slot):
            pltpu.make_async_copy(srcp.at[pl.ds(0, K)], src_v.at[slot],
                                  si.at[slot]).wait()
            pltpu.make_async_copy(dstp.at[pl.ds(0, K)], dst_v.at[slot],
                                  si.at[slot]).wait()

        def is_deg(j):
            return lax.select(c == 0, j < half, j >= half)

        if do_deg:
            pltpu.sync_copy(onesrow, ones_v)

        for p in range(NQ // NC):
            q = NC * p + c
            tbl = tblq.at[q]
            pltpu.sync_copy(zrows, agg_sh.at[pl.ds(rbase, RPT)])
            if do_deg and p == 0:
                pltpu.sync_copy(zdeg, deg_sh.at[pl.ds(rbase, RPT)])
            for b in range(NIDX):
                fire_idx(b, b)
            plsc.subcore_barrier()

            deg_pass = do_deg and p == 0

            def body(g, _):
                for h in (0, 1):
                    for b in range(NBUF):
                        islot = NBUF * h + b
                        pslot = NBUF * (1 - h) + b
                        jh = NIDX * g + NBUF * h
                        nxt = jh + NBUF + b

                        def drain():
                            pltpu.make_async_copy(
                                rows_v.at[b], agg_sh.at[dst_v.at[islot]],
                                ss.at[b]).wait()
                            if deg_pass:
                                @pl.when(is_deg(jh - NBUF))
                                def _():
                                    pltpu.make_async_copy(
                                        ones_v, deg_sh.at[dst_v.at[islot]],
                                        sd.at[b]).wait()
                            @pl.when(nxt < n_chunks)
                            def _():
                                fire_idx(nxt, pslot)
                        if h == 0:
                            pl.when(g > 0)(drain)
                        else:
                            drain()
                        wait_idx(islot)
                        pltpu.async_copy(tbl.at[src_v.at[islot]],
                                         rows_v.at[b], sg.at[b])
                    for b in range(NBUF):
                        islot = NBUF * h + b
                        pltpu.make_async_copy(tbl.at[src_v.at[islot]],
                                              rows_v.at[b], sg.at[b]).wait()
                        pltpu.async_copy(rows_v.at[b],
                                         agg_sh.at[dst_v.at[islot]],
                                         ss.at[b], add=True)
                        if deg_pass:
                            @pl.when(is_deg(NIDX * g + NBUF * h))
                            def _():
                                pltpu.async_copy(ones_v,
                                                 deg_sh.at[dst_v.at[islot]],
                                                 sd.at[b], add=True)
                return ()

            lax.fori_loop(0, n_g, body, ())
            for b in range(NBUF):
                pltpu.make_async_copy(rows_v.at[b],
                                      agg_sh.at[dst_v.at[NBUF + b]],
                                      ss.at[b]).wait()
                if deg_pass:
                    @pl.when(c == 1)
                    def _():
                        pltpu.make_async_copy(ones_v,
                                              deg_sh.at[dst_v.at[NBUF + b]],
                                              sd.at[b]).wait()
            plsc.subcore_barrier()
            pltpu.sync_copy(agg_sh.at[pl.ds(rbase, RPT)],
                            outagg.at[q, pl.ds(rbase, RPT)])
            if deg_pass:
                pltpu.sync_copy(deg_sh.at[pl.ds(rbase, RPT)],
                                outdeg.at[c, pl.ds(rbase, RPT)])

    return kfn


def _tc1_body(agg_ref, deg_ref, x_ref, w1l_ref, w1r_ref, b1_ref,
              w2l_ref, w2r_ref, b2_ref, pc_ref, r_ref):
    agg = jnp.concatenate([agg_ref[q] for q in range(NQ)], axis=1)
    degc = deg_ref[...]
    deg = jnp.maximum(degc[:, 0:1] + degc[:, 1:2], 1.0)
    mean = agg / deg
    cd = (((1,), (1,)), ((), ()))
    h = lax.dot_general(mean, w1l_ref[...], cd,
                        preferred_element_type=jnp.float32)
    h += lax.dot_general(x_ref[...], w1r_ref[...], cd,
                         preferred_element_type=jnp.float32)
    h = jnp.maximum(h + b1_ref[...], 0.0)
    p = lax.dot_general(h, w2l_ref[...], cd,
                        preferred_element_type=jnp.float32)
    r = lax.dot_general(h, w2r_ref[...], cd,
                        preferred_element_type=jnp.float32)
    for q in range(NQ):
        pc_ref[q] = p[:, q * DQ:(q + 1) * DQ]
    r_ref[...] = r + b2_ref[...]


def _tc2_body(agg_ref, deg_ref, r_ref, out_ref):
    agg = jnp.concatenate([agg_ref[q] for q in range(NQ)], axis=1)
    degc = deg_ref[...]
    deg = jnp.maximum(degc[:, 0:1] + degc[:, 1:2], 1.0)
    logits = agg / deg + r_ref[...]
    m = jnp.max(logits, axis=1, keepdims=True)
    sh = logits - m
    out_ref[...] = sh - jnp.log(jnp.sum(jnp.exp(sh), axis=1, keepdims=True))


def kernel(x, edge_index, W1l, W1r, b1, W2l, W2r, b2):
    n, d_in = x.shape
    e = edge_index.shape[1]
    d_h = W1l.shape[0]
    d_out = W2l.shape[0]
    n_acc = NS * RPT
    n_chunks = -(-e // (NS * K * NIDX)) * NIDX
    ep = NS * n_chunks * K
    src = jnp.concatenate([edge_index[0],
                           jnp.zeros((ep - e,), jnp.int32)])
    dst = jnp.concatenate([edge_index[1],
                           jnp.full((ep - e,), n, jnp.int32)])
    x_tbl = jnp.zeros((NQ, n_acc, DQ), jnp.float32)
    x_tbl = x_tbl.at[:, :n].set(x.reshape(n, NQ, DQ).transpose(1, 0, 2))
    zrows = jnp.zeros((RPT, DQ), jnp.float32)
    zdeg = jnp.zeros((RPT,), jnp.float32)
    onesrow = jnp.ones((K,), jnp.float32)

    agg1, deg2 = _sc_agg(n, n_acc, n_chunks, True)(
        x_tbl, src, dst, zrows, zdeg, onesrow)
    degT = deg2.T

    grid = -(-n // B_TC)
    pc, r = pl.pallas_call(
        _tc1_body,
        grid=(grid,),
        in_specs=[
            pl.BlockSpec((NQ, B_TC, DQ), lambda i: (0, i, 0)),
            pl.BlockSpec((B_TC, NC), lambda i: (i, 0)),
            pl.BlockSpec((B_TC, d_in), lambda i: (i, 0)),
            pl.BlockSpec((d_h, d_in), lambda i: (0, 0)),
            pl.BlockSpec((d_h, d_in), lambda i: (0, 0)),
            pl.BlockSpec((1, d_h), lambda i: (0, 0)),
            pl.BlockSpec((d_out, d_h), lambda i: (0, 0)),
            pl.BlockSpec((d_out, d_h), lambda i: (0, 0)),
            pl.BlockSpec((1, d_out), lambda i: (0, 0)),
        ],
        out_specs=[
            pl.BlockSpec((NQ, B_TC, DQ), lambda i: (0, i, 0)),
            pl.BlockSpec((B_TC, d_out), lambda i: (i, 0)),
        ],
        out_shape=[
            jax.ShapeDtypeStruct((NQ, n_acc, DQ), jnp.float32),
            jax.ShapeDtypeStruct((n, d_out), jnp.float32),
        ],
        compiler_params=pltpu.CompilerParams(
            dimension_semantics=("arbitrary",)),
    )(agg1, degT, x, W1l, W1r, b1.reshape(1, d_h), W2l, W2r,
      b2.reshape(1, d_out))

    agg2, = _sc_agg(n, n_acc, n_chunks, False)(
        pc, src, dst, zrows, zdeg, onesrow)

    out = pl.pallas_call(
        _tc2_body,
        grid=(grid,),
        in_specs=[
            pl.BlockSpec((NQ, B_TC, DQ), lambda i: (0, i, 0)),
            pl.BlockSpec((B_TC, NC), lambda i: (i, 0)),
            pl.BlockSpec((B_TC, d_out), lambda i: (i, 0)),
        ],
        out_specs=pl.BlockSpec((B_TC, d_out), lambda i: (i, 0)),
        out_shape=jax.ShapeDtypeStruct((n, d_out), jnp.float32),
        compiler_params=pltpu.CompilerParams(
            dimension_semantics=("arbitrary",)),
    )(agg2, degT, r)
    return out


# speedup vs baseline: 1.0022x; 1.0022x over previous
import functools

import jax
import jax.numpy as jnp
from jax import lax
from jax.experimental import pallas as pl
from jax.experimental.pallas import tpu as pltpu
from jax.experimental.pallas import tpu_sc as plsc

NS = 16
NC = 2
NQ = 2
K = 128
DQ = 128
RPT = 640
B_TC = 1024
NBUF = 2
NIDX = 2 * NBUF


@functools.lru_cache(maxsize=None)
def _sc_agg(n_rows, n_acc, n_chunks, do_deg):
    assert n_chunks % NIDX == 0
    n_g = n_chunks // NIDX
    half = n_chunks // 2
    out_ty = [jax.ShapeDtypeStruct((NQ, n_acc, DQ), jnp.float32)]
    if do_deg:
        out_ty.append(jax.ShapeDtypeStruct((NC, n_acc), jnp.float32))
    mesh = plsc.VectorSubcoreMesh(core_axis_name="c", subcore_axis_name="s")

    @functools.partial(
        pl.kernel,
        out_type=out_ty,
        mesh=mesh,
        scratch_types=[
            pltpu.VMEM((NIDX, 2, K), jnp.int32),
            pltpu.VMEM((NBUF, K, DQ), jnp.float32),
            pltpu.VMEM((K,), jnp.float32),
            pltpu.VMEM_SHARED((n_acc, DQ), jnp.float32),
            pltpu.VMEM_SHARED((n_acc,), jnp.float32),
            pltpu.SemaphoreType.DMA((NIDX,)),
            pltpu.SemaphoreType.DMA((NBUF,)),
            pltpu.SemaphoreType.DMA((NBUF,)),
            pltpu.SemaphoreType.DMA((NBUF,)),
        ],
    )
    def kfn(tblq, esc, zrows, zdeg, onesrow, *refs):
        if do_deg:
            (outagg, outdeg, idx_v, rows_v, ones_v, agg_sh,
             deg_sh, si, sg, ss, sd) = refs
        else:
            (outagg, idx_v, rows_v, ones_v, agg_sh,
             deg_sh, si, sg, ss, sd) = refs
        c = lax.axis_index("c")
        s = lax.axis_index("s")
        rbase = s * RPT
        cbase = s * n_chunks

        def fire_idx(j, slot):
            pltpu.async_copy(esc.at[cbase + j], idx_v.at[slot],
                             si.at[slot])

        def wait_idx(slot):
            pltpu.make_async_copy(esc.at[cbase], idx_v.at[slot],
                                  si.at[slot]).wait()

        def is_deg(j):
            return lax.select(c == 0, j < half, j >= half)

        if do_deg:
            pltpu.sync_copy(onesrow, ones_v)

        for p in range(NQ // NC):
            q = NC * p + c
            tbl = tblq.at[q]
            pltpu.sync_copy(zrows, agg_sh.at[pl.ds(rbase, RPT)])
            if do_deg and p == 0:
                pltpu.sync_copy(zdeg, deg_sh.at[pl.ds(rbase, RPT)])
            for b in range(NIDX):
                fire_idx(b, b)
            plsc.subcore_barrier()

            deg_pass = do_deg and p == 0

            def body(g, _):
                for h in (0, 1):
                    for b in range(NBUF):
                        islot = NBUF * h + b
                        pslot = NBUF * (1 - h) + b
                        jh = NIDX * g + NBUF * h
                        nxt = jh + NBUF + b

                        def drain():
                            pltpu.make_async_copy(
                                rows_v.at[b], agg_sh.at[idx_v.at[islot, 1]],
                                ss.at[b]).wait()
                            if deg_pass:
                                @pl.when(is_deg(jh - NBUF))
                                def _():
                                    pltpu.make_async_copy(
                                        ones_v, deg_sh.at[idx_v.at[islot, 1]],
                                        sd.at[b]).wait()
                            @pl.when(nxt < n_chunks)
                            def _():
                                fire_idx(nxt, pslot)
                        if h == 0:
                            pl.when(g > 0)(drain)
                        else:
                            drain()
                        wait_idx(islot)
                        pltpu.async_copy(tbl.at[idx_v.at[islot, 0]],
                                         rows_v.at[b], sg.at[b])
                    for b in range(NBUF):
                        islot = NBUF * h + b
                        pltpu.make_async_copy(tbl.at[idx_v.at[islot, 0]],
                                              rows_v.at[b], sg.at[b]).wait()
                        pltpu.async_copy(rows_v.at[b],
                                         agg_sh.at[idx_v.at[islot, 1]],
                                         ss.at[b], add=True)
                        if deg_pass:
                            @pl.when(is_deg(NIDX * g + NBUF * h))
                            def _():
                                pltpu.async_copy(ones_v,
                                                 deg_sh.at[idx_v.at[islot, 1]],
                                                 sd.at[b], add=True)
                return ()

            lax.fori_loop(0, n_g, body, ())
            for b in range(NBUF):
                pltpu.make_async_copy(rows_v.at[b],
                                      agg_sh.at[idx_v.at[NBUF + b, 1]],
                                      ss.at[b]).wait()
                if deg_pass:
                    @pl.when(c == 1)
                    def _():
                        pltpu.make_async_copy(ones_v,
                                              deg_sh.at[idx_v.at[NBUF + b, 1]],
                                              sd.at[b]).wait()
            plsc.subcore_barrier()
            pltpu.sync_copy(agg_sh.at[pl.ds(rbase, RPT)],
                            outagg.at[q, pl.ds(rbase, RPT)])
            if deg_pass:
                pltpu.sync_copy(deg_sh.at[pl.ds(rbase, RPT)],
                                outdeg.at[c, pl.ds(rbase, RPT)])

    return kfn


def _tc1_body(agg_ref, deg_ref, x_ref, w1l_ref, w1r_ref, b1_ref,
              w2l_ref, w2r_ref, b2_ref, pc_ref, r_ref):
    agg = jnp.concatenate([agg_ref[q] for q in range(NQ)], axis=1)
    degc = deg_ref[...]
    deg = jnp.maximum(degc[:, 0:1] + degc[:, 1:2], 1.0)
    mean = agg / deg
    cd = (((1,), (1,)), ((), ()))
    h = lax.dot_general(mean, w1l_ref[...], cd,
                        preferred_element_type=jnp.float32)
    h += lax.dot_general(x_ref[...], w1r_ref[...], cd,
                         preferred_element_type=jnp.float32)
    h = jnp.maximum(h + b1_ref[...], 0.0)
    p = lax.dot_general(h, w2l_ref[...], cd,
                        preferred_element_type=jnp.float32)
    r = lax.dot_general(h, w2r_ref[...], cd,
                        preferred_element_type=jnp.float32)
    for q in range(NQ):
        pc_ref[q] = p[:, q * DQ:(q + 1) * DQ]
    r_ref[...] = r + b2_ref[...]


def _tc2_body(agg_ref, deg_ref, r_ref, out_ref):
    agg = jnp.concatenate([agg_ref[q] for q in range(NQ)], axis=1)
    degc = deg_ref[...]
    deg = jnp.maximum(degc[:, 0:1] + degc[:, 1:2], 1.0)
    logits = agg / deg + r_ref[...]
    m = jnp.max(logits, axis=1, keepdims=True)
    sh = logits - m
    out_ref[...] = sh - jnp.log(jnp.sum(jnp.exp(sh), axis=1, keepdims=True))


def kernel(x, edge_index, W1l, W1r, b1, W2l, W2r, b2):
    n, d_in = x.shape
    e = edge_index.shape[1]
    d_h = W1l.shape[0]
    d_out = W2l.shape[0]
    n_acc = NS * RPT
    n_chunks = -(-e // (NS * K * NIDX)) * NIDX
    ep = NS * n_chunks * K
    src = jnp.concatenate([edge_index[0],
                           jnp.zeros((ep - e,), jnp.int32)])
    dst = jnp.concatenate([edge_index[1],
                           jnp.full((ep - e,), n, jnp.int32)])
    esc = (jnp.stack([src, dst]).reshape(2, ep // K, K)
           .transpose(1, 0, 2))
    x_tbl = jnp.zeros((NQ, n_acc, DQ), jnp.float32)
    x_tbl = x_tbl.at[:, :n].set(x.reshape(n, NQ, DQ).transpose(1, 0, 2))
    zrows = jnp.zeros((RPT, DQ), jnp.float32)
    zdeg = jnp.zeros((RPT,), jnp.float32)
    onesrow = jnp.ones((K,), jnp.float32)

    agg1, deg2 = _sc_agg(n, n_acc, n_chunks, True)(
        x_tbl, esc, zrows, zdeg, onesrow)
    degT = deg2.T

    grid = -(-n // B_TC)
    pc, r = pl.pallas_call(
        _tc1_body,
        grid=(grid,),
        in_specs=[
            pl.BlockSpec((NQ, B_TC, DQ), lambda i: (0, i, 0)),
            pl.BlockSpec((B_TC, NC), lambda i: (i, 0)),
            pl.BlockSpec((B_TC, d_in), lambda i: (i, 0)),
            pl.BlockSpec((d_h, d_in), lambda i: (0, 0)),
            pl.BlockSpec((d_h, d_in), lambda i: (0, 0)),
            pl.BlockSpec((1, d_h), lambda i: (0, 0)),
            pl.BlockSpec((d_out, d_h), lambda i: (0, 0)),
            pl.BlockSpec((d_out, d_h), lambda i: (0, 0)),
            pl.BlockSpec((1, d_out), lambda i: (0, 0)),
        ],
        out_specs=[
            pl.BlockSpec((NQ, B_TC, DQ), lambda i: (0, i, 0)),
            pl.BlockSpec((B_TC, d_out), lambda i: (i, 0)),
        ],
        out_shape=[
            jax.ShapeDtypeStruct((NQ, n_acc, DQ), jnp.float32),
            jax.ShapeDtypeStruct((n, d_out), jnp.float32),
        ],
        compiler_params=pltpu.CompilerParams(
            dimension_semantics=("arbitrary",)),
    )(agg1, degT, x, W1l, W1r, b1.reshape(1, d_h), W2l, W2r,
      b2.reshape(1, d_out))

    agg2, = _sc_agg(n, n_acc, n_chunks, False)(
        pc, esc, zrows, zdeg, onesrow)

    out = pl.pallas_call(
        _tc2_body,
        grid=(grid,),
        in_specs=[
            pl.BlockSpec((NQ, B_TC, DQ), lambda i: (0, i, 0)),
            pl.BlockSpec((B_TC, NC), lambda i: (i, 0)),
            pl.BlockSpec((B_TC, d_out), lambda i: (i, 0)),
        ],
        out_specs=pl.BlockSpec((B_TC, d_out), lambda i: (i, 0)),
        out_shape=jax.ShapeDtypeStruct((n, d_out), jnp.float32),
        compiler_params=pltpu.CompilerParams(
            dimension_semantics=("arbitrary",)),
    )(agg2, degT, r)
    return out
